# baseline (speedup 1.0000x reference)
import jax
import jax.numpy as jnp
from jax import lax
from jax.experimental import pallas as pl
from jax.experimental.pallas import tpu as pltpu
from jax.experimental.pallas import tpu_sc as plsc

_DEP_DIM = 4
_SLOT_NUM = 4096
_HASH_SEED = 1
_N_WORKERS = 32
_CHUNK_ROWS = 16
_CHUNK_ELEMS = _CHUNK_ROWS * _SLOT_NUM
_HASH_MULT = -1640531535


def _sc_body(out_hbm, buf):
    c = lax.axis_index("c")
    s = lax.axis_index("s")
    wid = s * 2 + c
    total_rows = _DEP_DIM * 4096
    rows_per_w = total_rows // _N_WORKERS
    n_chunks = rows_per_w // _CHUNK_ROWS

    lane = lax.iota(jnp.int32, 16)
    zeros16 = jnp.zeros((16,), jnp.float32)
    ones16 = jnp.ones((16,), jnp.float32)
    shift16 = jnp.full((16,), 16, jnp.int32)
    shift12 = jnp.full((16,), 12, jnp.int32)

    def zero_body(i, carry):
        for j in range(16):
            buf[pl.ds((i * 16 + j) * 16, 16)] = zeros16
        return carry

    lax.fori_loop(0, _CHUNK_ELEMS // 256, zero_body, 0)

    base = wid * rows_per_w

    def chunk_body(ci, carry):
        row0 = base + ci * _CHUNK_ROWS
        r = row0 + lane
        d = lax.shift_right_logical(r, shift12)
        b = jnp.bitwise_and(r, 4095)
        k = b * 4 + d
        m = jnp.bitwise_and(k, 4095) * 4 + lax.shift_right_logical(k, shift12)
        h = m * jnp.int32(_HASH_MULT) + jnp.int32(_HASH_SEED)
        h = jnp.bitwise_xor(h, lax.shift_right_logical(h, shift16))
        slot = jnp.bitwise_and(h, _SLOT_NUM - 1)
        flat = lane * _SLOT_NUM + slot
        plsc.store_scatter(buf, [flat], ones16)
        pltpu.sync_copy(buf, out_hbm.at[pl.ds(row0 * _SLOT_NUM, _CHUNK_ELEMS)])
        plsc.store_scatter(buf, [flat], zeros16)
        return carry

    lax.fori_loop(0, n_chunks, chunk_body, 0)


def kernel(input_tensor):
    batch_size = input_tensor.shape[0]
    sc_fn = pl.kernel(
        _sc_body,
        out_type=jax.ShapeDtypeStruct((_DEP_DIM * batch_size * _SLOT_NUM,),
                                      jnp.float32),
        scratch_types=[pltpu.VMEM((_CHUNK_ELEMS,), jnp.float32)],
        mesh=plsc.VectorSubcoreMesh(core_axis_name="c", subcore_axis_name="s"),
        compiler_params=pltpu.CompilerParams(needs_layout_passes=False),
    )
    out1d = sc_fn()
    return out1d.reshape(_DEP_DIM, batch_size, _SLOT_NUM)


# speedup vs baseline: 1.0006x; 1.0006x over previous
import jax
import jax.numpy as jnp
from jax import lax
from jax.experimental import pallas as pl
from jax.experimental.pallas import tpu as pltpu
from jax.experimental.pallas import tpu_sc as plsc

_DEP_DIM = 4
_SLOT_NUM = 4096
_HASH_SEED = 1
_N_WORKERS = 32
_HALF_ROWS = 8
_HALF_ELEMS = _HALF_ROWS * _SLOT_NUM
_HASH_MULT = -1640531535


def _slots_for(row0, lane, shift12, shift16):
    r = row0 + lane
    d = lax.shift_right_logical(r, shift12)
    b = jnp.bitwise_and(r, 4095)
    k = b * 4 + d
    m = jnp.bitwise_and(k, 4095) * 4 + lax.shift_right_logical(k, shift12)
    h = m * jnp.int32(_HASH_MULT) + jnp.int32(_HASH_SEED)
    h = jnp.bitwise_xor(h, lax.shift_right_logical(h, shift16))
    return jnp.bitwise_and(h, _SLOT_NUM - 1)


def _sc_body(out_hbm, buf, sem_a, sem_b):
    c = lax.axis_index("c")
    s = lax.axis_index("s")
    wid = s * 2 + c
    total_rows = _DEP_DIM * 4096
    rows_per_w = total_rows // _N_WORKERS
    n_pairs = rows_per_w // (2 * _HALF_ROWS)

    lane = lax.iota(jnp.int32, 16)
    zeros16 = jnp.zeros((16,), jnp.float32)
    ones16 = jnp.ones((16,), jnp.float32)
    shift16 = jnp.full((16,), 16, jnp.int32)
    shift12 = jnp.full((16,), 12, jnp.int32)
    mask_lo = lane < 8

    def zero_body(i, carry):
        for j in range(16):
            buf[pl.ds((i * 16 + j) * 16, 16)] = zeros16
        return carry

    lax.fori_loop(0, 2 * _HALF_ELEMS // 256, zero_body, 0)

    base = wid * rows_per_w

    def wait_half(sem):
        pltpu.make_async_copy(
            buf.at[pl.ds(0, _HALF_ELEMS)],
            out_hbm.at[pl.ds(base * _SLOT_NUM, _HALF_ELEMS)],
            sem,
        ).wait()

    def pair_body(ci, carry):
        pflat_a, pflat_b = carry
        row0 = base + ci * 2 * _HALF_ROWS

        slot_a = _slots_for(row0, lane, shift12, shift16)
        flat_a = lane * _SLOT_NUM + slot_a

        @pl.when(ci > 0)
        def _():
            wait_half(sem_a)
            plsc.store_scatter(buf, [pflat_a], zeros16, mask=mask_lo)

        plsc.store_scatter(buf, [flat_a], ones16, mask=mask_lo)
        pltpu.async_copy(
            buf.at[pl.ds(0, _HALF_ELEMS)],
            out_hbm.at[pl.ds(row0 * _SLOT_NUM, _HALF_ELEMS)],
            sem_a,
        )

        slot_b = _slots_for(row0 + _HALF_ROWS, lane, shift12, shift16)
        flat_b = _HALF_ELEMS + lane * _SLOT_NUM + slot_b

        @pl.when(ci > 0)
        def _():
            wait_half(sem_b)
            plsc.store_scatter(buf, [pflat_b], zeros16, mask=mask_lo)

        plsc.store_scatter(buf, [flat_b], ones16, mask=mask_lo)
        pltpu.async_copy(
            buf.at[pl.ds(_HALF_ELEMS, _HALF_ELEMS)],
            out_hbm.at[pl.ds((row0 + _HALF_ROWS) * _SLOT_NUM, _HALF_ELEMS)],
            sem_b,
        )
        return (flat_a, flat_b)

    init = (jnp.zeros((16,), jnp.int32), jnp.zeros((16,), jnp.int32))
    lax.fori_loop(0, n_pairs, pair_body, init)
    wait_half(sem_a)
    wait_half(sem_b)


def kernel(input_tensor):
    batch_size = input_tensor.shape[0]
    sc_fn = pl.kernel(
        _sc_body,
        out_type=jax.ShapeDtypeStruct((_DEP_DIM * batch_size * _SLOT_NUM,),
                                      jnp.float32),
        scratch_types=[
            pltpu.VMEM((2 * _HALF_ELEMS,), jnp.float32),
            pltpu.SemaphoreType.DMA,
            pltpu.SemaphoreType.DMA,
        ],
        mesh=plsc.VectorSubcoreMesh(core_axis_name="c", subcore_axis_name="s"),
        compiler_params=pltpu.CompilerParams(needs_layout_passes=False),
    )
    out1d = sc_fn()
    return out1d.reshape(_DEP_DIM, batch_size, _SLOT_NUM)


# speedup vs baseline: 4.5566x; 4.5540x over previous
import functools

import jax
import jax.numpy as jnp
from jax.experimental import pallas as pl

_DEP_DIM = 4
_SLOT_NUM = 4096
_HASH_SEED = 1
_BLOCK_ROWS = 256


def _onehot_kernel(out_ref, *, batch_size, block_rows):
    i = pl.program_id(0)
    r = jax.lax.broadcasted_iota(jnp.int32, (block_rows, 1), 0) + i * block_rows
    d = r // batch_size
    b = r % batch_size
    k = b * _DEP_DIM + d
    m = (k % batch_size) * _DEP_DIM + (k // batch_size)
    h = m.astype(jnp.uint32) * jnp.uint32(2654435761) + jnp.uint32(_HASH_SEED)
    h = h ^ (h >> jnp.uint32(16))
    s = (h % jnp.uint32(_SLOT_NUM)).astype(jnp.int32)
    slots = jax.lax.broadcasted_iota(jnp.int32, (block_rows, _SLOT_NUM), 1)
    out_ref[:, :] = (slots == s).astype(jnp.float32)


def kernel(input_tensor):
    batch_size = input_tensor.shape[0]
    total_rows = _DEP_DIM * batch_size
    out2d = pl.pallas_call(
        functools.partial(_onehot_kernel, batch_size=batch_size,
                          block_rows=_BLOCK_ROWS),
        grid=(total_rows // _BLOCK_ROWS,),
        out_specs=pl.BlockSpec((_BLOCK_ROWS, _SLOT_NUM), lambda i: (i, 0)),
        out_shape=jax.ShapeDtypeStruct((total_rows, _SLOT_NUM), jnp.float32),
    )()
    return out2d.reshape(_DEP_DIM, batch_size, _SLOT_NUM)


# speedup vs baseline: 4.6261x; 1.0153x over previous
import functools

import jax
import jax.numpy as jnp
from jax.experimental import pallas as pl

_DEP_DIM = 4
_SLOT_NUM = 4096
_HASH_SEED = 1
_BLOCK_B = 256


def _onehot_kernel(out_ref, *, batch_size, block_b):
    d = pl.program_id(0)
    ib = pl.program_id(1)
    b = jax.lax.broadcasted_iota(jnp.int32, (block_b, 1), 0) + ib * block_b
    k = b * _DEP_DIM + d
    m = (k % batch_size) * _DEP_DIM + (k // batch_size)
    h = m.astype(jnp.uint32) * jnp.uint32(2654435761) + jnp.uint32(_HASH_SEED)
    h = h ^ (h >> jnp.uint32(16))
    s = (h % jnp.uint32(_SLOT_NUM)).astype(jnp.int32)
    slots = jax.lax.broadcasted_iota(jnp.int32, (block_b, _SLOT_NUM), 1)
    out_ref[0, :, :] = (slots == s).astype(jnp.float32)


def kernel(input_tensor):
    batch_size = input_tensor.shape[0]
    grid = (_DEP_DIM, batch_size // _BLOCK_B)
    return pl.pallas_call(
        functools.partial(_onehot_kernel, batch_size=batch_size,
                          block_b=_BLOCK_B),
        grid=grid,
        out_specs=pl.BlockSpec((1, _BLOCK_B, _SLOT_NUM), lambda d, i: (d, i, 0)),
        out_shape=jax.ShapeDtypeStruct((_DEP_DIM, batch_size, _SLOT_NUM),
                                       jnp.float32),
    )()
